# baseline (speedup 1.0000x reference)
import functools

import jax
import jax.numpy as jnp
from jax import lax
from jax.experimental import pallas as pl
from jax.experimental.pallas import tpu as pltpu
from jax.experimental.pallas import tpu_sc as plsc

NC = 2
NS = 16
LANE = 128


def _mesh():
    return plsc.VectorSubcoreMesh(
        core_axis_name="c", subcore_axis_name="s", num_cores=NC, num_subcores=NS
    )


def _make_deg_kernel(n_pad, nb):
    rows = n_pad // NS
    out_rows = n_pad // (NC * NS)

    @functools.partial(
        pl.kernel,
        out_type=jax.ShapeDtypeStruct((n_pad,), jnp.float32),
        mesh=_mesh(),
        scratch_types=[
            pltpu.VMEM((nb, BS), jnp.int32),
            pltpu.VMEM((BS,), jnp.float32),
            pltpu.VMEM((rows,), jnp.float32),
            pltpu.VMEM_SHARED((n_pad,), jnp.float32),
        ],
    )
    def deg_kernel(dst_hbm, deg_hbm, dst_v, ones_v, zer_v, acc_sh):
        c = lax.axis_index("c")
        s = lax.axis_index("s")

        def fill_ones(i, carry):
            ones_v[pl.ds(i * 16, 16)] = jnp.full((16,), 1.0, jnp.float32)
            return carry

        lax.fori_loop(0, BS // 16, fill_ones, 0)

        def fill_zeros(i, carry):
            zer_v[pl.ds(i * 16, 16)] = jnp.zeros((16,), jnp.float32)
            return carry

        lax.fori_loop(0, rows // 16, fill_zeros, 0)

        pltpu.sync_copy(dst_hbm.at[s], dst_v)
        pltpu.sync_copy(zer_v, acc_sh.at[pl.ds(s * rows, rows)])
        plsc.subcore_barrier()

        def body(j, carry):
            pltpu.sync_copy(ones_v, acc_sh.at[dst_v.at[j]], add=True)
            return carry

        lax.fori_loop(0, nb, body, 0)
        plsc.subcore_barrier()

        off = (c * NS + s) * out_rows
        pltpu.sync_copy(acc_sh.at[pl.ds(off, out_rows)], zer_v.at[pl.ds(0, out_rows)])
        pltpu.sync_copy(zer_v.at[pl.ds(0, out_rows)], deg_hbm.at[pl.ds(off, out_rows)])

    return deg_kernel


NBUF = 2
BS = 128
NSEG = 2


def _make_prop_kernel(n, n_pad, nb, dc):
    cpc = dc // NC
    rows = n_pad // NS
    nh = nb // NSEG
    ngroups = nh // NBUF
    nz = rows // BS
    rem = rows - nz * BS

    scratch = (
        [pltpu.VMEM((nh, BS), jnp.int32), pltpu.VMEM((nh, BS), jnp.int32)]
        + [pltpu.VMEM((BS, LANE), jnp.float32) for _ in range(NBUF)]
        + [pltpu.VMEM_SHARED((n_pad, LANE), jnp.float32)]
        + [pltpu.SemaphoreType.DMA for _ in range(NBUF)]
    )

    @functools.partial(
        pl.kernel,
        out_type=jax.ShapeDtypeStruct((dc, n_pad, LANE), jnp.float32),
        mesh=_mesh(),
        scratch_types=scratch,
    )
    def prop_kernel(y_hbm, src_hbm, dst_hbm, acc_hbm, src_v, dst_v, *rest):
        bufs = rest[:NBUF]
        acc_sh = rest[NBUF]
        sems = rest[NBUF + 1:]
        c = lax.axis_index("c")
        s = lax.axis_index("s")
        base = s * rows

        for ci in range(cpc):
            chunk = c * cpc + ci

            def fill_zeros(t, carry):
                i = t // (LANE // 16)
                k = t % (LANE // 16)
                bufs[0][i, pl.ds(k * 16, 16)] = jnp.zeros((16,), jnp.float32)
                return carry

            lax.fori_loop(0, BS * (LANE // 16), fill_zeros, 0)
            for z in range(nz):
                pltpu.sync_copy(bufs[0], acc_sh.at[pl.ds(base + z * BS, BS)])
            if rem:
                pltpu.sync_copy(
                    bufs[0].at[pl.ds(0, rem)],
                    acc_sh.at[pl.ds(base + nz * BS, rem)],
                )
            plsc.subcore_barrier()

            for h in range(NSEG):
                pltpu.sync_copy(src_hbm.at[s].at[pl.ds(h * nh, nh)], src_v)
                pltpu.sync_copy(dst_hbm.at[s].at[pl.ds(h * nh, nh)], dst_v)
                for b in range(NBUF):
                    pltpu.async_copy(
                        y_hbm.at[chunk].at[src_v.at[b]], bufs[b], sems[b]
                    )

                def group(p, carry):
                    for b in range(NBUF):
                        j = p * NBUF + b
                        pltpu.make_async_copy(
                            y_hbm.at[chunk].at[src_v.at[j]], bufs[b], sems[b]
                        ).wait()
                        pltpu.sync_copy(bufs[b], acc_sh.at[dst_v.at[j]], add=True)
                        pltpu.async_copy(
                            y_hbm.at[chunk].at[src_v.at[j + NBUF]], bufs[b], sems[b]
                        )
                    return carry

                lax.fori_loop(0, ngroups - 1, group, 0)

                for b in range(NBUF):
                    j = (ngroups - 1) * NBUF + b
                    pltpu.make_async_copy(
                        y_hbm.at[chunk].at[src_v.at[j]], bufs[b], sems[b]
                    ).wait()
                    pltpu.sync_copy(bufs[b], acc_sh.at[dst_v.at[j]], add=True)

            plsc.subcore_barrier()
            pltpu.sync_copy(
                acc_sh.at[pl.ds(base, rows)], acc_hbm.at[chunk].at[pl.ds(base, rows)]
            )
            if ci + 1 < cpc:
                plsc.subcore_barrier()

    return prop_kernel


def _scaled_matmul(x, dinv2, w, rb=1000):
    n, din = x.shape
    dout = w.shape[1]
    dcn = dout // LANE
    grid = n // rb

    def body(x_ref, d_ref, w_ref, y_ref):
        z = x_ref[...] * d_ref[...]
        y = jnp.dot(z, w_ref[...], preferred_element_type=jnp.float32)
        for i in range(dcn):
            y_ref[i] = y[:, i * LANE:(i + 1) * LANE]

    return pl.pallas_call(
        body,
        grid=(grid,),
        in_specs=[
            pl.BlockSpec((rb, din), lambda i: (i, 0)),
            pl.BlockSpec((rb, 1), lambda i: (i, 0)),
            pl.BlockSpec((din, dout), lambda i: (0, 0)),
        ],
        out_specs=pl.BlockSpec((dcn, rb, LANE), lambda i: (0, i, 0)),
        out_shape=jax.ShapeDtypeStruct((dcn, n, LANE), jnp.float32),
    )(x, dinv2, w)


def _mid_layer(acc, y, dinv2, b2d, w, rb=1000):
    dci, n_pad, _ = acc.shape
    n = y.shape[1]
    din = dci * LANE
    dout = w.shape[1]
    dcn = dout // LANE
    grid = n // rb

    def body(a_ref, y_ref, d_ref, b_ref, w_ref, o_ref):
        a = jnp.concatenate([a_ref[i] for i in range(dci)], axis=1)
        yv = jnp.concatenate([y_ref[i] for i in range(dci)], axis=1)
        h = jnp.tanh((a + yv) * d_ref[...] + b_ref[...])
        o = jnp.dot(h * d_ref[...], w_ref[...], preferred_element_type=jnp.float32)
        for i in range(dcn):
            o_ref[i] = o[:, i * LANE:(i + 1) * LANE]

    return pl.pallas_call(
        body,
        grid=(grid,),
        in_specs=[
            pl.BlockSpec((dci, rb, LANE), lambda i: (0, i, 0)),
            pl.BlockSpec((dci, rb, LANE), lambda i: (0, i, 0)),
            pl.BlockSpec((rb, 1), lambda i: (i, 0)),
            pl.BlockSpec((1, din), lambda i: (0, 0)),
            pl.BlockSpec((din, dout), lambda i: (0, 0)),
        ],
        out_specs=pl.BlockSpec((dcn, rb, LANE), lambda i: (0, i, 0)),
        out_shape=jax.ShapeDtypeStruct((dcn, n, LANE), jnp.float32),
    )(acc, y, dinv2, b2d, w)


def _final_layer(acc, y, dinv2, b2d, rb=1000):
    dci, n_pad, _ = acc.shape
    n = y.shape[1]
    dout = dci * LANE
    grid = n // rb

    def body(a_ref, y_ref, d_ref, b_ref, o_ref):
        a = jnp.concatenate([a_ref[i] for i in range(dci)], axis=1)
        yv = jnp.concatenate([y_ref[i] for i in range(dci)], axis=1)
        o_ref[...] = (a + yv) * d_ref[...] + b_ref[...]

    return pl.pallas_call(
        body,
        grid=(grid,),
        in_specs=[
            pl.BlockSpec((dci, rb, LANE), lambda i: (0, i, 0)),
            pl.BlockSpec((dci, rb, LANE), lambda i: (0, i, 0)),
            pl.BlockSpec((rb, 1), lambda i: (i, 0)),
            pl.BlockSpec((1, dout), lambda i: (0, 0)),
        ],
        out_specs=pl.BlockSpec((rb, dout), lambda i: (i, 0)),
        out_shape=jax.ShapeDtypeStruct((n, dout), jnp.float32),
    )(acc, y, dinv2, b2d)


@jax.jit
def kernel(x, edge_index, W1, b1, W2, b2, W3, b3):
    n, din = x.shape
    e = edge_index.shape[1]
    dh = W1.shape[1]
    dout = W3.shape[1]

    per = e // NS
    unit = BS * NSEG * NBUF
    per_pad = ((per + unit - 1) // unit) * unit
    nb = per_pad // BS
    n_pad = ((n + 1 + 127) // 128) * 128

    src = edge_index[0].astype(jnp.int32).reshape(NS, per)
    dst = edge_index[1].astype(jnp.int32).reshape(NS, per)
    src16 = jnp.pad(src, ((0, 0), (0, per_pad - per))).reshape(NS, nb, BS)
    dst16 = jnp.pad(
        dst, ((0, 0), (0, per_pad - per)), constant_values=n
    ).reshape(NS, nb, BS)

    n_pad_deg = ((n // (NC * NS * 8)) + 1) * (NC * NS * 8)
    deg = _make_deg_kernel(n_pad_deg, nb)(dst16)
    dinv2 = lax.rsqrt(deg[:n] + 1.0)[:, None]

    prop_h = _make_prop_kernel(n, n_pad, nb, dh // LANE)
    prop_o = _make_prop_kernel(n, n_pad, nb, dout // LANE)

    y1 = _scaled_matmul(x, dinv2, W1)
    acc1 = prop_h(y1, src16, dst16)
    y2 = _mid_layer(acc1, y1, dinv2, b1.reshape(1, -1), W2)
    acc2 = prop_h(y2, src16, dst16)
    y3 = _mid_layer(acc2, y2, dinv2, b2.reshape(1, -1), W3)
    acc3 = prop_o(y3, src16, dst16)
    return _final_layer(acc3, y3, dinv2, b3.reshape(1, -1))


# speedup vs baseline: 1.0649x; 1.0649x over previous
import functools

import jax
import jax.numpy as jnp
from jax import lax
from jax.experimental import pallas as pl
from jax.experimental.pallas import tpu as pltpu
from jax.experimental.pallas import tpu_sc as plsc

NC = 2
NS = 16
LANE = 128


def _mesh():
    return plsc.VectorSubcoreMesh(
        core_axis_name="c", subcore_axis_name="s", num_cores=NC, num_subcores=NS
    )


def _make_deg_kernel(n_pad, nb):
    rows = n_pad // NS
    out_rows = n_pad // (NC * NS)

    @functools.partial(
        pl.kernel,
        out_type=jax.ShapeDtypeStruct((n_pad,), jnp.float32),
        mesh=_mesh(),
        scratch_types=[
            pltpu.VMEM((nb, BS), jnp.int32),
            pltpu.VMEM((BS,), jnp.float32),
            pltpu.VMEM((rows,), jnp.float32),
            pltpu.VMEM_SHARED((n_pad,), jnp.float32),
        ],
    )
    def deg_kernel(dst_hbm, deg_hbm, dst_v, ones_v, zer_v, acc_sh):
        c = lax.axis_index("c")
        s = lax.axis_index("s")

        def fill_ones(i, carry):
            ones_v[pl.ds(i * 16, 16)] = jnp.full((16,), 1.0, jnp.float32)
            return carry

        lax.fori_loop(0, BS // 16, fill_ones, 0)

        def fill_zeros(i, carry):
            zer_v[pl.ds(i * 16, 16)] = jnp.zeros((16,), jnp.float32)
            return carry

        lax.fori_loop(0, rows // 16, fill_zeros, 0)

        pltpu.sync_copy(dst_hbm.at[s], dst_v)
        pltpu.sync_copy(zer_v, acc_sh.at[pl.ds(s * rows, rows)])
        plsc.subcore_barrier()

        def body(j, carry):
            pltpu.sync_copy(ones_v, acc_sh.at[dst_v.at[j]], add=True)
            return carry

        lax.fori_loop(0, nb, body, 0)
        plsc.subcore_barrier()

        off = (c * NS + s) * out_rows
        pltpu.sync_copy(acc_sh.at[pl.ds(off, out_rows)], zer_v.at[pl.ds(0, out_rows)])
        pltpu.sync_copy(zer_v.at[pl.ds(0, out_rows)], deg_hbm.at[pl.ds(off, out_rows)])

    return deg_kernel


NBUF = 1
BS = 128


def _make_prop_kernel(n, n_pad, nb, dc):
    cpc = dc // NC
    rows = n_pad // NS
    ngroups = nb // NBUF
    nz = rows // BS
    rem = rows - nz * BS

    scratch = (
        [pltpu.VMEM((nb, BS), jnp.int32), pltpu.VMEM((nb, BS), jnp.int32)]
        + [pltpu.VMEM((BS, LANE), jnp.float32) for _ in range(NBUF)]
        + [pltpu.VMEM_SHARED((n_pad, LANE), jnp.float32)]
        + [pltpu.SemaphoreType.DMA for _ in range(NBUF)]
    )

    @functools.partial(
        pl.kernel,
        out_type=jax.ShapeDtypeStruct((dc, n_pad, LANE), jnp.float32),
        mesh=_mesh(),
        scratch_types=scratch,
    )
    def prop_kernel(y_hbm, src_hbm, dst_hbm, acc_hbm, src_v, dst_v, *rest):
        bufs = rest[:NBUF]
        acc_sh = rest[NBUF]
        sems = rest[NBUF + 1:]
        c = lax.axis_index("c")
        s = lax.axis_index("s")
        base = s * rows

        pltpu.sync_copy(src_hbm.at[s], src_v)
        pltpu.sync_copy(dst_hbm.at[s], dst_v)

        for ci in range(cpc):
            chunk = c * cpc + ci

            def fill_zeros(t, carry):
                i = t // (LANE // 16)
                k = t % (LANE // 16)
                bufs[0][i, pl.ds(k * 16, 16)] = jnp.zeros((16,), jnp.float32)
                return carry

            lax.fori_loop(0, BS * (LANE // 16), fill_zeros, 0)
            for z in range(nz):
                pltpu.sync_copy(bufs[0], acc_sh.at[pl.ds(base + z * BS, BS)])
            if rem:
                pltpu.sync_copy(
                    bufs[0].at[pl.ds(0, rem)],
                    acc_sh.at[pl.ds(base + nz * BS, rem)],
                )
            plsc.subcore_barrier()

            for b in range(NBUF):
                pltpu.async_copy(y_hbm.at[chunk].at[src_v.at[b]], bufs[b], sems[b])

            def group(p, carry):
                for b in range(NBUF):
                    j = p * NBUF + b
                    pltpu.make_async_copy(
                        y_hbm.at[chunk].at[src_v.at[j]], bufs[b], sems[b]
                    ).wait()
                    pltpu.sync_copy(bufs[b], acc_sh.at[dst_v.at[j]], add=True)
                    pltpu.async_copy(
                        y_hbm.at[chunk].at[src_v.at[j + NBUF]], bufs[b], sems[b]
                    )
                return carry

            lax.fori_loop(0, ngroups - 1, group, 0)

            for b in range(NBUF):
                j = (ngroups - 1) * NBUF + b
                pltpu.make_async_copy(
                    y_hbm.at[chunk].at[src_v.at[j]], bufs[b], sems[b]
                ).wait()
                pltpu.sync_copy(bufs[b], acc_sh.at[dst_v.at[j]], add=True)

            plsc.subcore_barrier()
            pltpu.sync_copy(
                acc_sh.at[pl.ds(base, rows)], acc_hbm.at[chunk].at[pl.ds(base, rows)]
            )
            if ci + 1 < cpc:
                plsc.subcore_barrier()

    return prop_kernel


def _scaled_matmul(x, dinv2, w, rb=2000):
    n, din = x.shape
    dout = w.shape[1]
    dcn = dout // LANE
    grid = n // rb

    def body(x_ref, d_ref, w_ref, y_ref):
        z = x_ref[...] * d_ref[...]
        y = jnp.dot(z, w_ref[...], preferred_element_type=jnp.float32)
        for i in range(dcn):
            y_ref[i] = y[:, i * LANE:(i + 1) * LANE]

    return pl.pallas_call(
        body,
        grid=(grid,),
        in_specs=[
            pl.BlockSpec((rb, din), lambda i: (i, 0)),
            pl.BlockSpec((rb, 1), lambda i: (i, 0)),
            pl.BlockSpec((din, dout), lambda i: (0, 0)),
        ],
        out_specs=pl.BlockSpec((dcn, rb, LANE), lambda i: (0, i, 0)),
        out_shape=jax.ShapeDtypeStruct((dcn, n, LANE), jnp.float32),
    )(x, dinv2, w)


def _mid_layer(acc, y, dinv2, b2d, w, rb=2000):
    dci, n_pad, _ = acc.shape
    n = y.shape[1]
    din = dci * LANE
    dout = w.shape[1]
    dcn = dout // LANE
    grid = n // rb

    def body(a_ref, y_ref, d_ref, b_ref, w_ref, o_ref):
        a = jnp.concatenate([a_ref[i] for i in range(dci)], axis=1)
        yv = jnp.concatenate([y_ref[i] for i in range(dci)], axis=1)
        h = jnp.tanh((a + yv) * d_ref[...] + b_ref[...])
        o = jnp.dot(h * d_ref[...], w_ref[...], preferred_element_type=jnp.float32)
        for i in range(dcn):
            o_ref[i] = o[:, i * LANE:(i + 1) * LANE]

    return pl.pallas_call(
        body,
        grid=(grid,),
        in_specs=[
            pl.BlockSpec((dci, rb, LANE), lambda i: (0, i, 0)),
            pl.BlockSpec((dci, rb, LANE), lambda i: (0, i, 0)),
            pl.BlockSpec((rb, 1), lambda i: (i, 0)),
            pl.BlockSpec((1, din), lambda i: (0, 0)),
            pl.BlockSpec((din, dout), lambda i: (0, 0)),
        ],
        out_specs=pl.BlockSpec((dcn, rb, LANE), lambda i: (0, i, 0)),
        out_shape=jax.ShapeDtypeStruct((dcn, n, LANE), jnp.float32),
    )(acc, y, dinv2, b2d, w)


def _final_layer(acc, y, dinv2, b2d, rb=2000):
    dci, n_pad, _ = acc.shape
    n = y.shape[1]
    dout = dci * LANE
    grid = n // rb

    def body(a_ref, y_ref, d_ref, b_ref, o_ref):
        a = jnp.concatenate([a_ref[i] for i in range(dci)], axis=1)
        yv = jnp.concatenate([y_ref[i] for i in range(dci)], axis=1)
        o_ref[...] = (a + yv) * d_ref[...] + b_ref[...]

    return pl.pallas_call(
        body,
        grid=(grid,),
        in_specs=[
            pl.BlockSpec((dci, rb, LANE), lambda i: (0, i, 0)),
            pl.BlockSpec((dci, rb, LANE), lambda i: (0, i, 0)),
            pl.BlockSpec((rb, 1), lambda i: (i, 0)),
            pl.BlockSpec((1, dout), lambda i: (0, 0)),
        ],
        out_specs=pl.BlockSpec((rb, dout), lambda i: (i, 0)),
        out_shape=jax.ShapeDtypeStruct((n, dout), jnp.float32),
    )(acc, y, dinv2, b2d)


@jax.jit
def kernel(x, edge_index, W1, b1, W2, b2, W3, b3):
    n, din = x.shape
    e = edge_index.shape[1]
    dh = W1.shape[1]
    dout = W3.shape[1]

    per = e // NS
    unit = BS * NBUF
    per_pad = ((per + unit - 1) // unit) * unit
    nb = per_pad // BS
    n_pad = ((n // 256) + 1) * 256

    src = edge_index[0].astype(jnp.int32).reshape(NS, per)
    dst = edge_index[1].astype(jnp.int32).reshape(NS, per)
    src16 = jnp.pad(src, ((0, 0), (0, per_pad - per))).reshape(NS, nb, BS)
    dst16 = jnp.pad(
        dst, ((0, 0), (0, per_pad - per)), constant_values=n
    ).reshape(NS, nb, BS)

    n_pad_deg = ((n // (NC * NS * 8)) + 1) * (NC * NS * 8)
    deg = _make_deg_kernel(n_pad_deg, nb)(dst16)
    dinv2 = lax.rsqrt(deg[:n] + 1.0)[:, None]

    prop_h = _make_prop_kernel(n, n_pad, nb, dh // LANE)
    prop_o = _make_prop_kernel(n, n_pad, nb, dout // LANE)

    y1 = _scaled_matmul(x, dinv2, W1)
    acc1 = prop_h(y1, src16, dst16)
    y2 = _mid_layer(acc1, y1, dinv2, b1.reshape(1, -1), W2)
    acc2 = prop_h(y2, src16, dst16)
    y3 = _mid_layer(acc2, y2, dinv2, b2.reshape(1, -1), W3)
    acc3 = prop_o(y3, src16, dst16)
    return _final_layer(acc3, y3, dinv2, b3.reshape(1, -1))


# speedup vs baseline: 1.0716x; 1.0063x over previous
import functools

import jax
import jax.numpy as jnp
from jax import lax
from jax.experimental import pallas as pl
from jax.experimental.pallas import tpu as pltpu
from jax.experimental.pallas import tpu_sc as plsc

NC = 2
NS = 16
LANE = 128


def _mesh():
    return plsc.VectorSubcoreMesh(
        core_axis_name="c", subcore_axis_name="s", num_cores=NC, num_subcores=NS
    )


def _make_deg_kernel(n_pad, nb):
    rows = n_pad // NS
    out_rows = n_pad // (NC * NS)

    @functools.partial(
        pl.kernel,
        out_type=jax.ShapeDtypeStruct((n_pad,), jnp.float32),
        mesh=_mesh(),
        scratch_types=[
            pltpu.VMEM((nb, BS), jnp.int32),
            pltpu.VMEM((BS,), jnp.float32),
            pltpu.VMEM((rows,), jnp.float32),
            pltpu.VMEM_SHARED((n_pad,), jnp.float32),
        ],
    )
    def deg_kernel(dst_hbm, deg_hbm, dst_v, ones_v, zer_v, acc_sh):
        c = lax.axis_index("c")
        s = lax.axis_index("s")

        def fill_ones(i, carry):
            ones_v[pl.ds(i * 16, 16)] = jnp.full((16,), 1.0, jnp.float32)
            return carry

        lax.fori_loop(0, BS // 16, fill_ones, 0)

        def fill_zeros(i, carry):
            zer_v[pl.ds(i * 16, 16)] = jnp.zeros((16,), jnp.float32)
            return carry

        lax.fori_loop(0, rows // 16, fill_zeros, 0)

        pltpu.sync_copy(dst_hbm.at[s], dst_v)
        pltpu.sync_copy(zer_v, acc_sh.at[pl.ds(s * rows, rows)])
        plsc.subcore_barrier()

        def body(j, carry):
            pltpu.sync_copy(ones_v, acc_sh.at[dst_v.at[j]], add=True)
            return carry

        lax.fori_loop(0, nb, body, 0)
        plsc.subcore_barrier()

        off = (c * NS + s) * out_rows
        pltpu.sync_copy(acc_sh.at[pl.ds(off, out_rows)], zer_v.at[pl.ds(0, out_rows)])
        pltpu.sync_copy(zer_v.at[pl.ds(0, out_rows)], deg_hbm.at[pl.ds(off, out_rows)])

    return deg_kernel


BS = 128
GB = 256


def _make_prop_kernel(n, n_pad, nb, dc):
    cpc = dc // NC
    rows = n_pad // NS
    per_sub = nb * BS
    gb = GB // BS
    half = per_sub // 2
    nbh = nb // 2
    ngroups = half // GB
    nz = rows // BS
    rem = rows - nz * BS

    scratch = (
        [pltpu.VMEM((half,), jnp.int32), pltpu.VMEM((nbh, BS), jnp.int32)]
        + [pltpu.VMEM((GB, LANE), jnp.float32)]
        + [pltpu.VMEM_SHARED((n_pad, LANE), jnp.float32)]
        + [pltpu.SemaphoreType.DMA]
    )

    @functools.partial(
        pl.kernel,
        out_type=jax.ShapeDtypeStruct((dc, n_pad, LANE), jnp.float32),
        mesh=_mesh(),
        scratch_types=scratch,
    )
    def prop_kernel(y_hbm, src_hbm, dst_hbm, acc_hbm, src_v, dst_v, buf, acc_sh, sem):
        c = lax.axis_index("c")
        s = lax.axis_index("s")
        base = s * rows

        for ci in range(cpc):
            chunk = c * cpc + ci

            def fill_zeros(t, carry):
                i = t // (LANE // 16)
                k = t % (LANE // 16)
                buf[i, pl.ds(k * 16, 16)] = jnp.zeros((16,), jnp.float32)
                return carry

            lax.fori_loop(0, BS * (LANE // 16), fill_zeros, 0)
            for z in range(nz):
                pltpu.sync_copy(
                    buf.at[pl.ds(0, BS)], acc_sh.at[pl.ds(base + z * BS, BS)]
                )
            if rem:
                pltpu.sync_copy(
                    buf.at[pl.ds(0, rem)],
                    acc_sh.at[pl.ds(base + nz * BS, rem)],
                )
            plsc.subcore_barrier()

            for h in range(2):
                pltpu.sync_copy(src_hbm.at[s].at[pl.ds(h * half, half)], src_v)
                pltpu.sync_copy(dst_hbm.at[s].at[pl.ds(h * nbh, nbh)], dst_v)

                def group(g, carry):
                    pltpu.sync_copy(
                        y_hbm.at[chunk].at[src_v.at[pl.ds(g * GB, GB)]], buf
                    )
                    for b in range(gb):
                        pltpu.sync_copy(
                            buf.at[pl.ds(b * BS, BS)],
                            acc_sh.at[dst_v.at[g * gb + b]],
                            add=True,
                        )
                    return carry

                lax.fori_loop(0, ngroups, group, 0)

            plsc.subcore_barrier()
            pltpu.sync_copy(
                acc_sh.at[pl.ds(base, rows)], acc_hbm.at[chunk].at[pl.ds(base, rows)]
            )
            if ci + 1 < cpc:
                plsc.subcore_barrier()

    return prop_kernel


def _scaled_matmul(x, dinv2, w, rb=2000):
    n, din = x.shape
    dout = w.shape[1]
    dcn = dout // LANE
    grid = n // rb

    def body(x_ref, d_ref, w_ref, y_ref):
        z = x_ref[...] * d_ref[...]
        y = jnp.dot(z, w_ref[...], preferred_element_type=jnp.float32)
        for i in range(dcn):
            y_ref[i] = y[:, i * LANE:(i + 1) * LANE]

    return pl.pallas_call(
        body,
        grid=(grid,),
        in_specs=[
            pl.BlockSpec((rb, din), lambda i: (i, 0)),
            pl.BlockSpec((rb, 1), lambda i: (i, 0)),
            pl.BlockSpec((din, dout), lambda i: (0, 0)),
        ],
        out_specs=pl.BlockSpec((dcn, rb, LANE), lambda i: (0, i, 0)),
        out_shape=jax.ShapeDtypeStruct((dcn, n, LANE), jnp.float32),
    )(x, dinv2, w)


def _mid_layer(acc, y, dinv2, b2d, w, rb=2000):
    dci, n_pad, _ = acc.shape
    n = y.shape[1]
    din = dci * LANE
    dout = w.shape[1]
    dcn = dout // LANE
    grid = n // rb

    def body(a_ref, y_ref, d_ref, b_ref, w_ref, o_ref):
        a = jnp.concatenate([a_ref[i] for i in range(dci)], axis=1)
        yv = jnp.concatenate([y_ref[i] for i in range(dci)], axis=1)
        h = jnp.tanh((a + yv) * d_ref[...] + b_ref[...])
        o = jnp.dot(h * d_ref[...], w_ref[...], preferred_element_type=jnp.float32)
        for i in range(dcn):
            o_ref[i] = o[:, i * LANE:(i + 1) * LANE]

    return pl.pallas_call(
        body,
        grid=(grid,),
        in_specs=[
            pl.BlockSpec((dci, rb, LANE), lambda i: (0, i, 0)),
            pl.BlockSpec((dci, rb, LANE), lambda i: (0, i, 0)),
            pl.BlockSpec((rb, 1), lambda i: (i, 0)),
            pl.BlockSpec((1, din), lambda i: (0, 0)),
            pl.BlockSpec((din, dout), lambda i: (0, 0)),
        ],
        out_specs=pl.BlockSpec((dcn, rb, LANE), lambda i: (0, i, 0)),
        out_shape=jax.ShapeDtypeStruct((dcn, n, LANE), jnp.float32),
    )(acc, y, dinv2, b2d, w)


def _final_layer(acc, y, dinv2, b2d, rb=2000):
    dci, n_pad, _ = acc.shape
    n = y.shape[1]
    dout = dci * LANE
    grid = n // rb

    def body(a_ref, y_ref, d_ref, b_ref, o_ref):
        a = jnp.concatenate([a_ref[i] for i in range(dci)], axis=1)
        yv = jnp.concatenate([y_ref[i] for i in range(dci)], axis=1)
        o_ref[...] = (a + yv) * d_ref[...] + b_ref[...]

    return pl.pallas_call(
        body,
        grid=(grid,),
        in_specs=[
            pl.BlockSpec((dci, rb, LANE), lambda i: (0, i, 0)),
            pl.BlockSpec((dci, rb, LANE), lambda i: (0, i, 0)),
            pl.BlockSpec((rb, 1), lambda i: (i, 0)),
            pl.BlockSpec((1, dout), lambda i: (0, 0)),
        ],
        out_specs=pl.BlockSpec((rb, dout), lambda i: (i, 0)),
        out_shape=jax.ShapeDtypeStruct((n, dout), jnp.float32),
    )(acc, y, dinv2, b2d)


@jax.jit
def kernel(x, edge_index, W1, b1, W2, b2, W3, b3):
    n, din = x.shape
    e = edge_index.shape[1]
    dh = W1.shape[1]
    dout = W3.shape[1]

    per = e // NS
    unit = 2 * GB
    per_pad = ((per + unit - 1) // unit) * unit
    nb = per_pad // BS
    n_pad = ((n // 256) + 1) * 256

    src = edge_index[0].astype(jnp.int32).reshape(NS, per)
    dst = edge_index[1].astype(jnp.int32).reshape(NS, per)
    src16 = jnp.pad(src, ((0, 0), (0, per_pad - per)))
    dst16 = jnp.pad(
        dst, ((0, 0), (0, per_pad - per)), constant_values=n
    ).reshape(NS, nb, BS)

    n_pad_deg = ((n // (NC * NS * 8)) + 1) * (NC * NS * 8)
    deg = _make_deg_kernel(n_pad_deg, nb)(dst16)
    dinv2 = lax.rsqrt(deg[:n] + 1.0)[:, None]

    prop_h = _make_prop_kernel(n, n_pad, nb, dh // LANE)
    prop_o = _make_prop_kernel(n, n_pad, nb, dout // LANE)

    y1 = _scaled_matmul(x, dinv2, W1)
    acc1 = prop_h(y1, src16, dst16)
    y2 = _mid_layer(acc1, y1, dinv2, b1.reshape(1, -1), W2)
    acc2 = prop_h(y2, src16, dst16)
    y3 = _mid_layer(acc2, y2, dinv2, b2.reshape(1, -1), W3)
    acc3 = prop_o(y3, src16, dst16)
    return _final_layer(acc3, y3, dinv2, b3.reshape(1, -1))


# speedup vs baseline: 1.0718x; 1.0002x over previous
import functools

import jax
import jax.numpy as jnp
from jax import lax
from jax.experimental import pallas as pl
from jax.experimental.pallas import tpu as pltpu
from jax.experimental.pallas import tpu_sc as plsc

NC = 2
NS = 16
LANE = 128


def _mesh():
    return plsc.VectorSubcoreMesh(
        core_axis_name="c", subcore_axis_name="s", num_cores=NC, num_subcores=NS
    )


def _make_deg_kernel(n_pad, nb):
    rows = n_pad // NS
    out_rows = n_pad // (NC * NS)

    @functools.partial(
        pl.kernel,
        out_type=jax.ShapeDtypeStruct((n_pad,), jnp.float32),
        mesh=_mesh(),
        scratch_types=[
            pltpu.VMEM((nb, BS), jnp.int32),
            pltpu.VMEM((BS,), jnp.float32),
            pltpu.VMEM((rows,), jnp.float32),
            pltpu.VMEM_SHARED((n_pad,), jnp.float32),
        ],
    )
    def deg_kernel(dst_hbm, deg_hbm, dst_v, ones_v, zer_v, acc_sh):
        c = lax.axis_index("c")
        s = lax.axis_index("s")

        def fill_ones(i, carry):
            ones_v[pl.ds(i * 16, 16)] = jnp.full((16,), 1.0, jnp.float32)
            return carry

        lax.fori_loop(0, BS // 16, fill_ones, 0)

        def fill_zeros(i, carry):
            zer_v[pl.ds(i * 16, 16)] = jnp.zeros((16,), jnp.float32)
            return carry

        lax.fori_loop(0, rows // 16, fill_zeros, 0)

        pltpu.sync_copy(dst_hbm.at[s], dst_v)
        pltpu.sync_copy(zer_v, acc_sh.at[pl.ds(s * rows, rows)])
        plsc.subcore_barrier()

        def body(j, carry):
            pltpu.sync_copy(ones_v, acc_sh.at[dst_v.at[j]], add=True)
            return carry

        lax.fori_loop(0, nb, body, 0)
        plsc.subcore_barrier()

        off = (c * NS + s) * out_rows
        pltpu.sync_copy(acc_sh.at[pl.ds(off, out_rows)], zer_v.at[pl.ds(0, out_rows)])
        pltpu.sync_copy(zer_v.at[pl.ds(0, out_rows)], deg_hbm.at[pl.ds(off, out_rows)])

    return deg_kernel


BS = 128
GB = 256


def _make_prop_kernel(n, n_pad, nb, dc):
    cpc = dc // NC
    rows = n_pad // NS
    per_sub = nb * BS
    gb = GB // BS
    half = per_sub // 2
    nbh = nb // 2
    ngroups = half // GB
    nz = rows // BS
    rem = rows - nz * BS

    scratch = (
        [pltpu.VMEM((half,), jnp.int32), pltpu.VMEM((nbh, BS), jnp.int32)]
        + [pltpu.VMEM((GB, LANE), jnp.float32)]
        + [pltpu.VMEM_SHARED((n_pad, LANE), jnp.float32)]
        + [pltpu.SemaphoreType.DMA]
    )

    @functools.partial(
        pl.kernel,
        out_type=jax.ShapeDtypeStruct((dc, n_pad, LANE), jnp.float32),
        mesh=_mesh(),
        scratch_types=scratch,
    )
    def prop_kernel(y_hbm, src_hbm, dst_hbm, acc_hbm, src_v, dst_v, buf, acc_sh, sem):
        c = lax.axis_index("c")
        s = lax.axis_index("s")
        base = s * rows

        for ci in range(cpc):
            chunk = c * cpc + ci

            def fill_zeros(t, carry):
                i = t // (LANE // 16)
                k = t % (LANE // 16)
                buf[i, pl.ds(k * 16, 16)] = jnp.zeros((16,), jnp.float32)
                return carry

            lax.fori_loop(0, BS * (LANE // 16), fill_zeros, 0)
            for z in range(nz):
                pltpu.sync_copy(
                    buf.at[pl.ds(0, BS)], acc_sh.at[pl.ds(base + z * BS, BS)]
                )
            if rem:
                pltpu.sync_copy(
                    buf.at[pl.ds(0, rem)],
                    acc_sh.at[pl.ds(base + nz * BS, rem)],
                )
            plsc.subcore_barrier()

            for h in range(2):
                pltpu.sync_copy(src_hbm.at[s].at[pl.ds(h * half, half)], src_v)
                pltpu.sync_copy(dst_hbm.at[s].at[pl.ds(h * nbh, nbh)], dst_v)

                def group(g, carry):
                    pltpu.sync_copy(
                        y_hbm.at[chunk].at[src_v.at[pl.ds(g * GB, GB)]], buf
                    )
                    for b in range(gb):
                        pltpu.sync_copy(
                            buf.at[pl.ds(b * BS, BS)],
                            acc_sh.at[dst_v.at[g * gb + b]],
                            add=True,
                        )
                    return carry

                lax.fori_loop(0, ngroups, group, 0)

            plsc.subcore_barrier()
            pltpu.sync_copy(
                acc_sh.at[pl.ds(base, rows)], acc_hbm.at[chunk].at[pl.ds(base, rows)]
            )
            if ci + 1 < cpc:
                plsc.subcore_barrier()

    return prop_kernel


def _scale_chunk(x, dinv2, rb=2000):
    n, din = x.shape
    dcn = din // LANE
    grid = n // rb

    def body(x_ref, d_ref, z_ref):
        z = x_ref[...] * d_ref[...]
        for i in range(dcn):
            z_ref[i] = z[:, i * LANE:(i + 1) * LANE]

    return pl.pallas_call(
        body,
        grid=(grid,),
        in_specs=[
            pl.BlockSpec((rb, din), lambda i: (i, 0)),
            pl.BlockSpec((rb, 1), lambda i: (i, 0)),
        ],
        out_specs=pl.BlockSpec((dcn, rb, LANE), lambda i: (0, i, 0)),
        out_shape=jax.ShapeDtypeStruct((dcn, n, LANE), jnp.float32),
    )(x, dinv2)


def _first_layer(acc, z, dinv2, b2d, w1, w2, rb=2000):
    dci, n_pad, _ = acc.shape
    n = z.shape[1]
    din = dci * LANE
    dout = w2.shape[1]
    dcn = dout // LANE
    grid = n // rb

    def body(a_ref, z_ref, d_ref, b_ref, w1_ref, w2_ref, o_ref):
        a = jnp.concatenate([a_ref[i] for i in range(dci)], axis=1)
        zv = jnp.concatenate([z_ref[i] for i in range(dci)], axis=1)
        p = (a + zv) * d_ref[...]
        h = jnp.tanh(
            jnp.dot(p, w1_ref[...], preferred_element_type=jnp.float32) + b_ref[...]
        )
        o = jnp.dot(h * d_ref[...], w2_ref[...], preferred_element_type=jnp.float32)
        for i in range(dcn):
            o_ref[i] = o[:, i * LANE:(i + 1) * LANE]

    dh = w1.shape[1]
    return pl.pallas_call(
        body,
        grid=(grid,),
        in_specs=[
            pl.BlockSpec((dci, rb, LANE), lambda i: (0, i, 0)),
            pl.BlockSpec((dci, rb, LANE), lambda i: (0, i, 0)),
            pl.BlockSpec((rb, 1), lambda i: (i, 0)),
            pl.BlockSpec((1, dh), lambda i: (0, 0)),
            pl.BlockSpec((din, dh), lambda i: (0, 0)),
            pl.BlockSpec((dh, dout), lambda i: (0, 0)),
        ],
        out_specs=pl.BlockSpec((dcn, rb, LANE), lambda i: (0, i, 0)),
        out_shape=jax.ShapeDtypeStruct((dcn, n, LANE), jnp.float32),
    )(acc, z, dinv2, b2d, w1, w2)


def _mid_layer(acc, y, dinv2, b2d, w, rb=2000):
    dci, n_pad, _ = acc.shape
    n = y.shape[1]
    din = dci * LANE
    dout = w.shape[1]
    dcn = dout // LANE
    grid = n // rb

    def body(a_ref, y_ref, d_ref, b_ref, w_ref, o_ref):
        a = jnp.concatenate([a_ref[i] for i in range(dci)], axis=1)
        yv = jnp.concatenate([y_ref[i] for i in range(dci)], axis=1)
        h = jnp.tanh((a + yv) * d_ref[...] + b_ref[...])
        o = jnp.dot(h * d_ref[...], w_ref[...], preferred_element_type=jnp.float32)
        for i in range(dcn):
            o_ref[i] = o[:, i * LANE:(i + 1) * LANE]

    return pl.pallas_call(
        body,
        grid=(grid,),
        in_specs=[
            pl.BlockSpec((dci, rb, LANE), lambda i: (0, i, 0)),
            pl.BlockSpec((dci, rb, LANE), lambda i: (0, i, 0)),
            pl.BlockSpec((rb, 1), lambda i: (i, 0)),
            pl.BlockSpec((1, din), lambda i: (0, 0)),
            pl.BlockSpec((din, dout), lambda i: (0, 0)),
        ],
        out_specs=pl.BlockSpec((dcn, rb, LANE), lambda i: (0, i, 0)),
        out_shape=jax.ShapeDtypeStruct((dcn, n, LANE), jnp.float32),
    )(acc, y, dinv2, b2d, w)


def _final_layer(acc, y, dinv2, b2d, rb=2000):
    dci, n_pad, _ = acc.shape
    n = y.shape[1]
    dout = dci * LANE
    grid = n // rb

    def body(a_ref, y_ref, d_ref, b_ref, o_ref):
        a = jnp.concatenate([a_ref[i] for i in range(dci)], axis=1)
        yv = jnp.concatenate([y_ref[i] for i in range(dci)], axis=1)
        o_ref[...] = (a + yv) * d_ref[...] + b_ref[...]

    return pl.pallas_call(
        body,
        grid=(grid,),
        in_specs=[
            pl.BlockSpec((dci, rb, LANE), lambda i: (0, i, 0)),
            pl.BlockSpec((dci, rb, LANE), lambda i: (0, i, 0)),
            pl.BlockSpec((rb, 1), lambda i: (i, 0)),
            pl.BlockSpec((1, dout), lambda i: (0, 0)),
        ],
        out_specs=pl.BlockSpec((rb, dout), lambda i: (i, 0)),
        out_shape=jax.ShapeDtypeStruct((n, dout), jnp.float32),
    )(acc, y, dinv2, b2d)


@jax.jit
def kernel(x, edge_index, W1, b1, W2, b2, W3, b3):
    n, din = x.shape
    e = edge_index.shape[1]
    dh = W1.shape[1]
    dout = W3.shape[1]

    per = e // NS
    unit = 2 * GB
    per_pad = ((per + unit - 1) // unit) * unit
    nb = per_pad // BS
    n_pad = ((n // 256) + 1) * 256

    src = edge_index[0].astype(jnp.int32).reshape(NS, per)
    dst = edge_index[1].astype(jnp.int32).reshape(NS, per)
    src16 = jnp.pad(src, ((0, 0), (0, per_pad - per)))
    dst16 = jnp.pad(
        dst, ((0, 0), (0, per_pad - per)), constant_values=n
    ).reshape(NS, nb, BS)

    n_pad_deg = ((n // (NC * NS * 8)) + 1) * (NC * NS * 8)
    deg = _make_deg_kernel(n_pad_deg, nb)(dst16)
    dinv2 = lax.rsqrt(deg[:n] + 1.0)[:, None]

    prop_in = _make_prop_kernel(n, n_pad, nb, din // LANE)
    prop_h = _make_prop_kernel(n, n_pad, nb, dh // LANE)
    prop_out = _make_prop_kernel(n, n_pad, nb, dout // LANE)

    z1 = _scale_chunk(x, dinv2)
    acc1 = prop_in(z1, src16, dst16)
    y2 = _first_layer(acc1, z1, dinv2, b1.reshape(1, -1), W1, W2)
    acc2 = prop_h(y2, src16, dst16)
    y3 = _mid_layer(acc2, y2, dinv2, b2.reshape(1, -1), W3)
    acc3 = prop_out(y3, src16, dst16)
    return _final_layer(acc3, y3, dinv2, b3.reshape(1, -1))


# speedup vs baseline: 1.3052x; 1.2178x over previous
import functools

import jax
import jax.numpy as jnp
from jax import lax
from jax.experimental import pallas as pl
from jax.experimental.pallas import tpu as pltpu
from jax.experimental.pallas import tpu_sc as plsc

NC = 2
NS = 16
LANE = 128


def _mesh():
    return plsc.VectorSubcoreMesh(
        core_axis_name="c", subcore_axis_name="s", num_cores=NC, num_subcores=NS
    )


def _make_deg_kernel(n_pad, nb):
    rows = n_pad // NS
    out_rows = n_pad // (NC * NS)

    @functools.partial(
        pl.kernel,
        out_type=jax.ShapeDtypeStruct((n_pad,), jnp.float32),
        mesh=_mesh(),
        scratch_types=[
            pltpu.VMEM((nb, BS), jnp.int32),
            pltpu.VMEM((BS,), jnp.float32),
            pltpu.VMEM((rows,), jnp.float32),
            pltpu.VMEM_SHARED((n_pad,), jnp.float32),
        ],
    )
    def deg_kernel(dst_hbm, deg_hbm, dst_v, ones_v, zer_v, acc_sh):
        c = lax.axis_index("c")
        s = lax.axis_index("s")

        def fill_ones(i, carry):
            ones_v[pl.ds(i * 16, 16)] = jnp.full((16,), 1.0, jnp.float32)
            return carry

        lax.fori_loop(0, BS // 16, fill_ones, 0)

        def fill_zeros(i, carry):
            zer_v[pl.ds(i * 16, 16)] = jnp.zeros((16,), jnp.float32)
            return carry

        lax.fori_loop(0, rows // 16, fill_zeros, 0)

        pltpu.sync_copy(dst_hbm.at[s], dst_v)
        pltpu.sync_copy(zer_v, acc_sh.at[pl.ds(s * rows, rows)])
        plsc.subcore_barrier()

        def body(j, carry):
            pltpu.sync_copy(ones_v, acc_sh.at[dst_v.at[j]], add=True)
            return carry

        lax.fori_loop(0, nb, body, 0)
        plsc.subcore_barrier()

        off = (c * NS + s) * out_rows
        pltpu.sync_copy(acc_sh.at[pl.ds(off, out_rows)], zer_v.at[pl.ds(0, out_rows)])
        pltpu.sync_copy(zer_v.at[pl.ds(0, out_rows)], deg_hbm.at[pl.ds(off, out_rows)])

    return deg_kernel


BS = 128


def _make_prop_kernel(n, n_pad, nb, dc):
    cpc = dc // NC
    rows = n_pad // NS
    nz = rows // BS
    rem = rows - nz * BS

    scratch = (
        [pltpu.VMEM((nb, BS), jnp.int32), pltpu.VMEM((nb, BS), jnp.int32)]
        + [pltpu.VMEM((BS, LANE), jnp.float32)]
        + [pltpu.VMEM_SHARED((n_pad, LANE), jnp.float32)]
        + [pltpu.SemaphoreType.DMA]
    )

    @functools.partial(
        pl.kernel,
        out_type=jax.ShapeDtypeStruct((dc, n_pad, LANE), jnp.float32),
        mesh=_mesh(),
        scratch_types=scratch,
    )
    def prop_kernel(y_hbm, src_hbm, dst_hbm, acc_hbm, src_v, dst_v, buf, acc_sh, sem):
        c = lax.axis_index("c")
        s = lax.axis_index("s")
        base = s * rows

        pltpu.sync_copy(src_hbm.at[s], src_v)
        pltpu.sync_copy(dst_hbm.at[s], dst_v)

        for ci in range(cpc):
            chunk = c * cpc + ci

            def fill_zeros(t, carry):
                i = t // (LANE // 16)
                k = t % (LANE // 16)
                buf[i, pl.ds(k * 16, 16)] = jnp.zeros((16,), jnp.float32)
                return carry

            lax.fori_loop(0, BS * (LANE // 16), fill_zeros, 0)
            for z in range(nz):
                pltpu.sync_copy(buf, acc_sh.at[pl.ds(base + z * BS, BS)])
            if rem:
                pltpu.sync_copy(
                    buf.at[pl.ds(0, rem)],
                    acc_sh.at[pl.ds(base + nz * BS, rem)],
                )
            plsc.subcore_barrier()

            pltpu.async_copy(y_hbm.at[chunk].at[src_v.at[0]], buf, sem)

            def batch(j, carry):
                pltpu.make_async_copy(
                    y_hbm.at[chunk].at[src_v.at[j]], buf, sem
                ).wait()
                pltpu.sync_copy(buf, acc_sh.at[dst_v.at[j]], add=True)
                pltpu.async_copy(y_hbm.at[chunk].at[src_v.at[j + 1]], buf, sem)
                return carry

            lax.fori_loop(0, nb - 1, batch, 0)

            pltpu.make_async_copy(
                y_hbm.at[chunk].at[src_v.at[nb - 1]], buf, sem
            ).wait()
            pltpu.sync_copy(buf, acc_sh.at[dst_v.at[nb - 1]], add=True)

            plsc.subcore_barrier()
            pltpu.sync_copy(
                acc_sh.at[pl.ds(base, rows)], acc_hbm.at[chunk].at[pl.ds(base, rows)]
            )
            if ci + 1 < cpc:
                plsc.subcore_barrier()

    return prop_kernel


def _scale_chunk(x, dinv2, rb=2000):
    n, din = x.shape
    dcn = din // LANE
    grid = n // rb

    def body(x_ref, d_ref, z_ref):
        z = x_ref[...] * d_ref[...]
        for i in range(dcn):
            z_ref[i] = z[:, i * LANE:(i + 1) * LANE]

    return pl.pallas_call(
        body,
        grid=(grid,),
        in_specs=[
            pl.BlockSpec((rb, din), lambda i: (i, 0)),
            pl.BlockSpec((rb, 1), lambda i: (i, 0)),
        ],
        out_specs=pl.BlockSpec((dcn, rb, LANE), lambda i: (0, i, 0)),
        out_shape=jax.ShapeDtypeStruct((dcn, n, LANE), jnp.float32),
    )(x, dinv2)


def _first_layer(acc, z, dinv2, b2d, w1, w2, rb=2000):
    dci, n_pad, _ = acc.shape
    n = z.shape[1]
    din = dci * LANE
    dout = w2.shape[1]
    dcn = dout // LANE
    grid = n // rb

    def body(a_ref, z_ref, d_ref, b_ref, w1_ref, w2_ref, o_ref):
        a = jnp.concatenate([a_ref[i] for i in range(dci)], axis=1)
        zv = jnp.concatenate([z_ref[i] for i in range(dci)], axis=1)
        p = (a + zv) * d_ref[...]
        h = jnp.tanh(
            jnp.dot(p, w1_ref[...], preferred_element_type=jnp.float32) + b_ref[...]
        )
        o = jnp.dot(h * d_ref[...], w2_ref[...], preferred_element_type=jnp.float32)
        for i in range(dcn):
            o_ref[i] = o[:, i * LANE:(i + 1) * LANE]

    dh = w1.shape[1]
    return pl.pallas_call(
        body,
        grid=(grid,),
        in_specs=[
            pl.BlockSpec((dci, rb, LANE), lambda i: (0, i, 0)),
            pl.BlockSpec((dci, rb, LANE), lambda i: (0, i, 0)),
            pl.BlockSpec((rb, 1), lambda i: (i, 0)),
            pl.BlockSpec((1, dh), lambda i: (0, 0)),
            pl.BlockSpec((din, dh), lambda i: (0, 0)),
            pl.BlockSpec((dh, dout), lambda i: (0, 0)),
        ],
        out_specs=pl.BlockSpec((dcn, rb, LANE), lambda i: (0, i, 0)),
        out_shape=jax.ShapeDtypeStruct((dcn, n, LANE), jnp.float32),
    )(acc, z, dinv2, b2d, w1, w2)


def _mid_layer(acc, y, dinv2, b2d, w, rb=2000):
    dci, n_pad, _ = acc.shape
    n = y.shape[1]
    din = dci * LANE
    dout = w.shape[1]
    dcn = dout // LANE
    grid = n // rb

    def body(a_ref, y_ref, d_ref, b_ref, w_ref, o_ref):
        a = jnp.concatenate([a_ref[i] for i in range(dci)], axis=1)
        yv = jnp.concatenate([y_ref[i] for i in range(dci)], axis=1)
        h = jnp.tanh((a + yv) * d_ref[...] + b_ref[...])
        o = jnp.dot(h * d_ref[...], w_ref[...], preferred_element_type=jnp.float32)
        for i in range(dcn):
            o_ref[i] = o[:, i * LANE:(i + 1) * LANE]

    return pl.pallas_call(
        body,
        grid=(grid,),
        in_specs=[
            pl.BlockSpec((dci, rb, LANE), lambda i: (0, i, 0)),
            pl.BlockSpec((dci, rb, LANE), lambda i: (0, i, 0)),
            pl.BlockSpec((rb, 1), lambda i: (i, 0)),
            pl.BlockSpec((1, din), lambda i: (0, 0)),
            pl.BlockSpec((din, dout), lambda i: (0, 0)),
        ],
        out_specs=pl.BlockSpec((dcn, rb, LANE), lambda i: (0, i, 0)),
        out_shape=jax.ShapeDtypeStruct((dcn, n, LANE), jnp.float32),
    )(acc, y, dinv2, b2d, w)


def _final_layer(acc, y, dinv2, b2d, rb=2000):
    dci, n_pad, _ = acc.shape
    n = y.shape[1]
    dout = dci * LANE
    grid = n // rb

    def body(a_ref, y_ref, d_ref, b_ref, o_ref):
        a = jnp.concatenate([a_ref[i] for i in range(dci)], axis=1)
        yv = jnp.concatenate([y_ref[i] for i in range(dci)], axis=1)
        o_ref[...] = (a + yv) * d_ref[...] + b_ref[...]

    return pl.pallas_call(
        body,
        grid=(grid,),
        in_specs=[
            pl.BlockSpec((dci, rb, LANE), lambda i: (0, i, 0)),
            pl.BlockSpec((dci, rb, LANE), lambda i: (0, i, 0)),
            pl.BlockSpec((rb, 1), lambda i: (i, 0)),
            pl.BlockSpec((1, dout), lambda i: (0, 0)),
        ],
        out_specs=pl.BlockSpec((rb, dout), lambda i: (i, 0)),
        out_shape=jax.ShapeDtypeStruct((n, dout), jnp.float32),
    )(acc, y, dinv2, b2d)


@jax.jit
def kernel(x, edge_index, W1, b1, W2, b2, W3, b3):
    n, din = x.shape
    e = edge_index.shape[1]
    dh = W1.shape[1]
    dout = W3.shape[1]

    per = e // NS
    per_pad = ((per + BS - 1) // BS) * BS
    nb = per_pad // BS
    n_pad = ((n // 256) + 1) * 256

    src = edge_index[0].astype(jnp.int32).reshape(NS, per)
    dst = edge_index[1].astype(jnp.int32).reshape(NS, per)
    src16 = jnp.pad(src, ((0, 0), (0, per_pad - per))).reshape(NS, nb, BS)
    dst16 = jnp.pad(
        dst, ((0, 0), (0, per_pad - per)), constant_values=n
    ).reshape(NS, nb, BS)

    n_pad_deg = ((n // (NC * NS * 8)) + 1) * (NC * NS * 8)
    deg = _make_deg_kernel(n_pad_deg, nb)(dst16)
    dinv2 = lax.rsqrt(deg[:n] + 1.0)[:, None]

    prop_in = _make_prop_kernel(n, n_pad, nb, din // LANE)
    prop_h = _make_prop_kernel(n, n_pad, nb, dh // LANE)
    prop_out = _make_prop_kernel(n, n_pad, nb, dout // LANE)

    z1 = _scale_chunk(x, dinv2)
    acc1 = prop_in(z1, src16, dst16)
    y2 = _first_layer(acc1, z1, dinv2, b1.reshape(1, -1), W1, W2)
    acc2 = prop_h(y2, src16, dst16)
    y3 = _mid_layer(acc2, y2, dinv2, b2.reshape(1, -1), W3)
    acc3 = prop_out(y3, src16, dst16)
    return _final_layer(acc3, y3, dinv2, b3.reshape(1, -1))
